# baseline (device time: 47960 ns/iter reference)
import jax
import jax.numpy as jnp
from jax import lax
from jax.experimental import pallas as pl
from jax.experimental.pallas import tpu as pltpu

N_DEV = 16
LOG2_N = 4
PARTNER_XOR = (8, 4, 1, 3)
B, SQ, D = 2, 128, 512
HQ_LOC, DH = 8, 64
SKV = 128
T = B * SQ
NC = 2
CR = T // NC


def _kv_prep(K_ext, V_ext, idx):
    k3 = K_ext.reshape(B, SKV, 128 * DH)
    v3 = V_ext.reshape(B, SKV, 128 * DH)
    n_pair = HQ_LOC // 2

    def prep_body(s_ref, k_in, v_in, k_out, v_out):
        kb = k_in[0]
        vb = v_in[0]
        k_out[0, 0] = kb[:, :DH].astype(jnp.bfloat16)
        k_out[0, 1] = kb[:, DH:].astype(jnp.bfloat16)
        v_out[0, 0] = vb[:, :DH].astype(jnp.bfloat16)
        v_out[0, 1] = vb[:, DH:].astype(jnp.bfloat16)

    grid_spec = pltpu.PrefetchScalarGridSpec(
        num_scalar_prefetch=1,
        grid=(B, n_pair),
        in_specs=[
            pl.BlockSpec((1, SKV, 2 * DH), lambda b, p, s: (b, 0, s[0] * n_pair + p)),
            pl.BlockSpec((1, SKV, 2 * DH), lambda b, p, s: (b, 0, s[0] * n_pair + p)),
        ],
        out_specs=[
            pl.BlockSpec((1, 2, SKV, DH), lambda b, p, s: (b, p, 0, 0)),
            pl.BlockSpec((1, 2, SKV, DH), lambda b, p, s: (b, p, 0, 0)),
        ],
    )
    return pl.pallas_call(
        prep_body,
        grid_spec=grid_spec,
        out_shape=[
            jax.ShapeDtypeStruct((B, HQ_LOC, SKV, DH), jnp.bfloat16),
            jax.ShapeDtypeStruct((B, HQ_LOC, SKV, DH), jnp.bfloat16),
        ],
    )(idx, k3, v3)


def kernel(x, Wq, Wo, K_ext, V_ext):
    my_i = lax.axis_index("i")

    k_sl, v_sl = _kv_prep(K_ext, V_ext, jnp.reshape(my_i, (1,)).astype(jnp.int32))

    def body(x_ref, wq_ref, wo_ref, k_ref, v_ref, out_ref,
             acc_ref, o_ref, recv_ref, send_sems, recv_sems):
        my = lax.axis_index("i")

        barrier = pltpu.get_barrier_semaphore()
        for step in range(LOG2_N):
            partner = my ^ PARTNER_XOR[step]
            pl.semaphore_signal(
                barrier, inc=1,
                device_id=(partner,), device_id_type=pl.DeviceIdType.MESH,
            )

        q2d = jnp.dot(
            x_ref[...].reshape(T, D), wq_ref[...],
            preferred_element_type=jnp.float32,
        ).astype(jnp.bfloat16)

        def attend(b):
            for h in range(HQ_LOC):
                q = q2d[b * SQ:(b + 1) * SQ, h * DH:(h + 1) * DH]
                kk = k_ref[b, h]
                vv = v_ref[b, h]
                s = lax.dot_general(
                    q, kk, (((1,), (1,)), ((), ())),
                    preferred_element_type=jnp.float32,
                ) * 0.125
                m = jnp.max(s, axis=-1, keepdims=True)
                p = jnp.exp(s - m)
                l = jnp.sum(p, axis=-1, keepdims=True)
                o = lax.dot_general(
                    p.astype(jnp.bfloat16), vv, (((1,), (0,)), ((), ())),
                    preferred_element_type=jnp.float32,
                )
                o_ref[b * SQ:(b + 1) * SQ, h * DH:(h + 1) * DH] = o / l
            acc_ref[b, :, :] = jnp.dot(
                o_ref[pl.ds(b * CR, CR), :], wo_ref[...],
                preferred_element_type=jnp.float32,
            ).astype(jnp.bfloat16)

        rdmas = {}

        def issue(step, c):
            partner = my ^ PARTNER_XOR[step]
            r = pltpu.make_async_remote_copy(
                src_ref=acc_ref.at[c],
                dst_ref=recv_ref.at[step, c],
                send_sem=send_sems.at[step, c],
                recv_sem=recv_sems.at[step, c],
                device_id=(partner,),
                device_id_type=pl.DeviceIdType.MESH,
            )
            r.start()
            rdmas[(step, c)] = r

        def finish(step, c):
            rdmas[(step, c)].wait()
            acc_ref[c, :, :] = acc_ref[c] + recv_ref[step, c]

        attend(0)
        pl.semaphore_wait(barrier, LOG2_N)
        issue(0, 0)
        attend(1)
        issue(0, 1)
        for step in range(LOG2_N):
            finish(step, 0)
            if step + 1 < LOG2_N:
                issue(step + 1, 0)
            else:
                out_ref[0, :, :] = acc_ref[0].astype(jnp.float32)
            finish(step, 1)
            if step + 1 < LOG2_N:
                issue(step + 1, 1)
        out_ref[1, :, :] = acc_ref[1].astype(jnp.float32)

    return pl.pallas_call(
        body,
        out_shape=jax.ShapeDtypeStruct((B, SQ, D), jnp.float32),
        in_specs=[pl.BlockSpec(memory_space=pltpu.VMEM)] * 5,
        out_specs=pl.BlockSpec(memory_space=pltpu.VMEM),
        scratch_shapes=[
            pltpu.VMEM((NC, CR, D), jnp.bfloat16),
            pltpu.VMEM((T, D), jnp.float32),
            pltpu.VMEM((LOG2_N, NC, CR, D), jnp.bfloat16),
            pltpu.SemaphoreType.DMA((LOG2_N, NC)),
            pltpu.SemaphoreType.DMA((LOG2_N, NC)),
        ],
        compiler_params=pltpu.CompilerParams(collective_id=0),
    )(x, Wq, Wo, k_sl, v_sl)


# device time: 37320 ns/iter; 1.2851x vs baseline; 1.2851x over previous
import jax
import jax.numpy as jnp
from jax import lax
from jax.experimental import pallas as pl
from jax.experimental.pallas import tpu as pltpu

N_DEV = 16
LOG2_N = 4
PARTNER_XOR = (8, 4, 1, 3)
B, SQ, D = 2, 128, 512
HQ_LOC, DH = 8, 64
SKV = 128
T = B * SQ
NC = 2
CR = T // NC


def kernel(x, Wq, Wo, K_ext, V_ext):
    my_i = lax.axis_index("i")

    k_sl = lax.dynamic_slice_in_dim(
        K_ext.reshape(B, SKV, 128 * DH), my_i * HQ_LOC * DH, HQ_LOC * DH, axis=2
    ).reshape(B, SKV, HQ_LOC, DH)
    v_sl = lax.dynamic_slice_in_dim(
        V_ext.reshape(B, SKV, 128 * DH), my_i * HQ_LOC * DH, HQ_LOC * DH, axis=2
    ).reshape(B, SKV, HQ_LOC, DH)
    k_sl = k_sl.transpose(0, 2, 1, 3).astype(jnp.bfloat16)
    v_sl = v_sl.transpose(0, 2, 1, 3).astype(jnp.bfloat16)

    def body(x_ref, wq_ref, wo_ref, k_ref, v_ref, out_ref,
             acc_ref, o_ref, recv_ref, send_sems, recv_sems):
        my = lax.axis_index("i")

        barrier = pltpu.get_barrier_semaphore()
        for step in range(LOG2_N):
            partner = my ^ PARTNER_XOR[step]
            pl.semaphore_signal(
                barrier, inc=1,
                device_id=(partner,), device_id_type=pl.DeviceIdType.MESH,
            )

        q2d = jnp.dot(
            x_ref[...].reshape(T, D), wq_ref[...],
            preferred_element_type=jnp.float32,
        ).astype(jnp.bfloat16)

        def attend(b):
            for h in range(HQ_LOC):
                q = q2d[b * SQ:(b + 1) * SQ, h * DH:(h + 1) * DH]
                kk = k_ref[b, h]
                vv = v_ref[b, h]
                s = lax.dot_general(
                    q, kk, (((1,), (1,)), ((), ())),
                    preferred_element_type=jnp.float32,
                ) * 0.125
                m = jnp.max(s, axis=-1, keepdims=True)
                p = jnp.exp(s - m)
                l = jnp.sum(p, axis=-1, keepdims=True)
                o = lax.dot_general(
                    p.astype(jnp.bfloat16), vv, (((1,), (0,)), ((), ())),
                    preferred_element_type=jnp.float32,
                )
                o_ref[b * SQ:(b + 1) * SQ, h * DH:(h + 1) * DH] = o / l
            acc_ref[b, :, :] = jnp.dot(
                o_ref[pl.ds(b * CR, CR), :], wo_ref[...],
                preferred_element_type=jnp.float32,
            ).astype(jnp.bfloat16)

        rdmas = {}

        def issue(step, c):
            partner = my ^ PARTNER_XOR[step]
            r = pltpu.make_async_remote_copy(
                src_ref=acc_ref.at[c],
                dst_ref=recv_ref.at[step, c],
                send_sem=send_sems.at[step, c],
                recv_sem=recv_sems.at[step, c],
                device_id=(partner,),
                device_id_type=pl.DeviceIdType.MESH,
            )
            r.start()
            rdmas[(step, c)] = r

        def finish(step, c):
            rdmas[(step, c)].wait()
            acc_ref[c, :, :] = acc_ref[c] + recv_ref[step, c]

        attend(0)
        pl.semaphore_wait(barrier, LOG2_N)
        issue(0, 0)
        attend(1)
        issue(0, 1)
        for step in range(LOG2_N):
            finish(step, 0)
            if step + 1 < LOG2_N:
                issue(step + 1, 0)
            else:
                out_ref[0, :, :] = acc_ref[0].astype(jnp.float32)
            finish(step, 1)
            if step + 1 < LOG2_N:
                issue(step + 1, 1)
        out_ref[1, :, :] = acc_ref[1].astype(jnp.float32)

    return pl.pallas_call(
        body,
        out_shape=jax.ShapeDtypeStruct((B, SQ, D), jnp.float32),
        in_specs=[pl.BlockSpec(memory_space=pltpu.VMEM)] * 5,
        out_specs=pl.BlockSpec(memory_space=pltpu.VMEM),
        scratch_shapes=[
            pltpu.VMEM((NC, CR, D), jnp.bfloat16),
            pltpu.VMEM((T, D), jnp.float32),
            pltpu.VMEM((LOG2_N, NC, CR, D), jnp.bfloat16),
            pltpu.SemaphoreType.DMA((LOG2_N, NC)),
            pltpu.SemaphoreType.DMA((LOG2_N, NC)),
        ],
        compiler_params=pltpu.CompilerParams(collective_id=0),
    )(x, Wq, Wo, k_sl, v_sl)


# device time: 34025 ns/iter; 1.4096x vs baseline; 1.0968x over previous
import jax
import jax.numpy as jnp
from jax import lax
from jax.experimental import pallas as pl
from jax.experimental.pallas import tpu as pltpu

N_DEV = 16
LOG2_N = 4
PARTNER_XOR = (8, 4, 1, 3)
B, SQ, D = 2, 128, 512
HQ_LOC, DH = 8, 64
SKV = 128
T = B * SQ
NC = 2
CR = T // NC


def kernel(x, Wq, Wo, K_ext, V_ext):
    my_i = lax.axis_index("i")

    k_sl = lax.dynamic_slice_in_dim(K_ext, my_i * HQ_LOC, HQ_LOC, axis=2)
    v_sl = lax.dynamic_slice_in_dim(V_ext, my_i * HQ_LOC, HQ_LOC, axis=2)
    k_sl = k_sl.transpose(0, 2, 1, 3).astype(jnp.bfloat16)
    v_sl = v_sl.transpose(0, 2, 1, 3).astype(jnp.bfloat16)

    def body(x_ref, wq_ref, wo_ref, k_ref, v_ref, out_ref,
             acc_ref, o_ref, recv_ref, send_sems, recv_sems):
        my = lax.axis_index("i")

        barrier = pltpu.get_barrier_semaphore()
        for step in range(LOG2_N):
            partner = my ^ PARTNER_XOR[step]
            pl.semaphore_signal(
                barrier, inc=1,
                device_id=(partner,), device_id_type=pl.DeviceIdType.MESH,
            )

        q2d = jnp.dot(
            x_ref[...].reshape(T, D), wq_ref[...],
            preferred_element_type=jnp.float32,
        ).astype(jnp.bfloat16)

        def attend(b):
            for h in range(HQ_LOC):
                q = q2d[b * SQ:(b + 1) * SQ, h * DH:(h + 1) * DH]
                kk = k_ref[b, h]
                vv = v_ref[b, h]
                s = lax.dot_general(
                    q, kk, (((1,), (1,)), ((), ())),
                    preferred_element_type=jnp.float32,
                ) * 0.125
                m = jnp.max(s, axis=-1, keepdims=True)
                p = jnp.exp(s - m)
                l = jnp.sum(p, axis=-1, keepdims=True)
                o = lax.dot_general(
                    p.astype(jnp.bfloat16), vv, (((1,), (0,)), ((), ())),
                    preferred_element_type=jnp.float32,
                )
                o_ref[b * SQ:(b + 1) * SQ, h * DH:(h + 1) * DH] = o / l
            acc_ref[b, :, :] = jnp.dot(
                o_ref[pl.ds(b * CR, CR), :], wo_ref[...],
                preferred_element_type=jnp.float32,
            ).astype(jnp.bfloat16)

        rdmas = {}

        def issue(step, c):
            partner = my ^ PARTNER_XOR[step]
            r = pltpu.make_async_remote_copy(
                src_ref=acc_ref.at[c],
                dst_ref=recv_ref.at[step, c],
                send_sem=send_sems.at[step, c],
                recv_sem=recv_sems.at[step, c],
                device_id=(partner,),
                device_id_type=pl.DeviceIdType.MESH,
            )
            r.start()
            rdmas[(step, c)] = r

        def finish(step, c):
            rdmas[(step, c)].wait()
            acc_ref[c, :, :] = acc_ref[c] + recv_ref[step, c]

        attend(0)
        pl.semaphore_wait(barrier, LOG2_N)
        issue(0, 0)
        attend(1)
        issue(0, 1)
        for step in range(LOG2_N):
            finish(step, 0)
            if step + 1 < LOG2_N:
                issue(step + 1, 0)
            else:
                out_ref[0, :, :] = acc_ref[0].astype(jnp.float32)
            finish(step, 1)
            if step + 1 < LOG2_N:
                issue(step + 1, 1)
        out_ref[1, :, :] = acc_ref[1].astype(jnp.float32)

    return pl.pallas_call(
        body,
        out_shape=jax.ShapeDtypeStruct((B, SQ, D), jnp.float32),
        in_specs=[pl.BlockSpec(memory_space=pltpu.VMEM)] * 5,
        out_specs=pl.BlockSpec(memory_space=pltpu.VMEM),
        scratch_shapes=[
            pltpu.VMEM((NC, CR, D), jnp.bfloat16),
            pltpu.VMEM((T, D), jnp.float32),
            pltpu.VMEM((LOG2_N, NC, CR, D), jnp.bfloat16),
            pltpu.SemaphoreType.DMA((LOG2_N, NC)),
            pltpu.SemaphoreType.DMA((LOG2_N, NC)),
        ],
        compiler_params=pltpu.CompilerParams(collective_id=0),
    )(x, Wq, Wo, k_sl, v_sl)
